# baseline (speedup 1.0000x reference)
import functools

import jax
import jax.numpy as jnp
from jax import lax
from jax.experimental import pallas as pl
from jax.experimental.pallas import tpu as pltpu
from jax.experimental.pallas import tpu_sc as plsc


def _pack_rows(y):
    w = lax.bitcast_convert_type(y, jnp.uint32)
    r = (w + 0x7FFF + ((w >> 16) & 1)) >> 16
    dw = y.shape[1] // 2
    packed = r[:, :dw] | (r[:, dw:] << 16)
    return lax.bitcast_convert_type(packed, jnp.int32)


def _unpack_rows(wi):
    w = lax.bitcast_convert_type(wi, jnp.uint32)
    lo = lax.bitcast_convert_type(w << 16, jnp.float32)
    hi = lax.bitcast_convert_type(w & jnp.uint32(0xFFFF0000), jnp.float32)
    return jnp.concatenate([lo, hi], axis=1)


def _proj_body(xa_ref, xb_ref, w_ref, b_ref, o_ref):
    w = w_ref[...]
    b = b_ref[...]
    ya = jnp.dot(xa_ref[...], w, preferred_element_type=jnp.float32) + b
    yb = jnp.dot(xb_ref[...], w, preferred_element_type=jnp.float32) + b
    o_ref[...] = jnp.concatenate([_pack_rows(ya), _pack_rows(yb)], axis=1)


def _project(x, w, b):
    n, fdim = x.shape
    d = w.shape[1]
    blk = 1000
    nb = n // 2 // blk
    assert (n // 2) % blk == 0
    folded = pl.pallas_call(
        _proj_body,
        grid=(nb,),
        in_specs=[
            pl.BlockSpec((blk, fdim), lambda i: (i, 0)),
            pl.BlockSpec((blk, fdim), lambda i, nb_=nb: (i + nb_, 0)),
            pl.BlockSpec((fdim, d), lambda i: (0, 0)),
            pl.BlockSpec((1, d), lambda i: (0, 0)),
        ],
        out_specs=pl.BlockSpec((blk, d), lambda i: (i, 0)),
        out_shape=jax.ShapeDtypeStruct((n // 2, d), jnp.int32),
    )(x, x, w, b.reshape(1, d))
    return folded.reshape(n, d // 2)


def _cast_body(x_ref, o_ref):
    x = x_ref[...]
    h = x.shape[0] // 2
    o_ref[...] = jnp.concatenate(
        [_pack_rows(x[:h]), _pack_rows(x[h:])], axis=1)


def _pack_table(x):
    n, d = x.shape
    folded = pl.pallas_call(
        _cast_body,
        out_shape=jax.ShapeDtypeStruct((n // 2, d), jnp.int32),
    )(x)
    return folded.reshape(n, d // 2)


def _remap_idx(idx, n):
    h = n // 2
    idx = idx.astype(jnp.int32)
    return jnp.where(idx < h, 2 * idx, 2 * idx - (n - 1))


def _make_sc_gather(Bn, d, s_per, streams):
    info = plsc.get_sparse_core_info()
    NW = info.num_cores * info.num_subcores
    CB = 16
    G = CB * s_per
    GH = G // 2
    rows_w = Bn // NW
    nchunks = rows_w // CB
    assert Bn % (NW * CB) == 0 and nchunks % 2 == 0
    mesh = plsc.VectorSubcoreMesh(core_axis_name="c", subcore_axis_name="s")
    ns = len(streams)

    dw = d // 2
    idx_scratch = [
        pltpu.VMEM((rows_w * (s_per if st["reduce"] else 1),), jnp.int32)
        for st in streams
    ]
    row_scratch = [
        pltpu.VMEM(((G if st["reduce"] else CB), dw), jnp.int32)
        for st in streams
    ] * 2
    acc_scratch = [pltpu.VMEM((CB, dw), jnp.int32)
                   for st in streams if st["reduce"]] * 2
    nred = sum(1 for st in streams if st["reduce"])

    @functools.partial(
        pl.kernel,
        mesh=mesh,
        out_type=[jax.ShapeDtypeStruct((Bn, dw), jnp.int32)] * ns,
        scratch_types=idx_scratch + row_scratch + acc_scratch
        + [pltpu.SemaphoreType.DMA] * 4,
        compiler_params=pltpu.CompilerParams(use_tc_tiling_on_sc=False,
                                             needs_layout_passes=False),
    )
    def sc_gather(*refs):
        tabs = refs[:ns]
        idxs = refs[ns:2 * ns]
        outs = refs[2 * ns:3 * ns]
        k = 3 * ns
        ix = refs[k:k + ns]
        rows = (refs[k + ns:k + 2 * ns], refs[k + 2 * ns:k + 3 * ns])
        k2 = k + 3 * ns
        accs = (refs[k2:k2 + nred], refs[k2 + nred:k2 + 2 * nred])
        sem_g = refs[k2 + 2 * nred:k2 + 2 * nred + 2]
        sem_o = refs[k2 + 2 * nred + 2:k2 + 2 * nred + 4]

        wid = lax.axis_index("s") * info.num_cores + lax.axis_index("c")
        base = wid * rows_w

        for t, st in enumerate(streams):
            rep = s_per if st["reduce"] else 1
            pltpu.sync_copy(idxs[t].at[pl.ds(base * rep, rows_w * rep)],
                            ix[t])

        def fire_gathers(c, s):
            for t, st in enumerate(streams):
                if st["reduce"]:
                    o = pl.multiple_of(c * G, 8)
                    pltpu.async_copy(
                        tabs[t].at[ix[t].at[pl.ds(o, GH)]],
                        rows[s][t].at[pl.ds(0, GH)], sem_g[s])
                    pltpu.async_copy(
                        tabs[t].at[ix[t].at[pl.ds(o + GH, GH)]],
                        rows[s][t].at[pl.ds(GH, GH)], sem_g[s])
                else:
                    o = pl.multiple_of(c * CB, CB)
                    pltpu.async_copy(tabs[t].at[ix[t].at[pl.ds(o, CB)]],
                                     rows[s][t], sem_g[s])

        def wait_gathers(s):
            for t, st in enumerate(streams):
                if st["reduce"]:
                    pltpu.make_async_copy(tabs[t].at[pl.ds(0, GH)],
                                          rows[s][t].at[pl.ds(0, GH)],
                                          sem_g[s]).wait()
                    pltpu.make_async_copy(tabs[t].at[pl.ds(0, GH)],
                                          rows[s][t].at[pl.ds(GH, GH)],
                                          sem_g[s]).wait()
                else:
                    pltpu.make_async_copy(tabs[t].at[pl.ds(0, CB)],
                                          rows[s][t], sem_g[s]).wait()

        def fire_out(c, s):
            off = pl.multiple_of(base + c * CB, CB)
            r = 0
            for t, st in enumerate(streams):
                src = accs[s][r] if st["reduce"] else rows[s][t]
                if st["reduce"]:
                    r += 1
                pltpu.async_copy(src, outs[t].at[pl.ds(off, CB)], sem_o[s])

        def drain_out(s):
            r = 0
            for t, st in enumerate(streams):
                dst = accs[s][r] if st["reduce"] else rows[s][t]
                if st["reduce"]:
                    r += 1
                pltpu.make_async_copy(outs[t].at[pl.ds(0, CB)], dst,
                                      sem_o[s]).wait()

        def reduce(rows_t, acc_t):
            bf = jnp.bfloat16

            def per_center(i, carry):
                rbase = i * s_per
                for g in range(dw // 16):
                    sl = pl.ds(g * 16, 16)
                    vals = [plsc.bitcast(rows_t[rbase + j, sl], bf)
                            for j in range(s_per)]
                    while len(vals) > 1:
                        nxt = [vals[k] + vals[k + 1]
                               for k in range(0, len(vals) - 1, 2)]
                        if len(vals) % 2:
                            nxt.append(vals[-1])
                        vals = nxt
                    acc_t[i, sl] = plsc.bitcast(vals[0], jnp.int32)
                return carry
            lax.fori_loop(0, CB, per_center, 0)

        def reduce_all(s):
            r = 0
            for t, st in enumerate(streams):
                if st["reduce"]:
                    reduce(rows[s][t], accs[s][r])
                    r += 1

        fire_gathers(0, 0)

        def step(cc, carry):
            for s in (0, 1):
                c = cc * 2 + s
                if s == 0:
                    @pl.when(cc > 0)
                    def _():
                        drain_out(1)
                    fire_gathers(c + 1, 1)
                else:
                    drain_out(0)

                    @pl.when(cc < nchunks // 2 - 1)
                    def _():
                        fire_gathers(c + 1, 0)
                wait_gathers(s)
                reduce_all(s)
                fire_out(c, s)
            return carry

        lax.fori_loop(0, nchunks // 2, step, 0)
        drain_out(1)

    return sc_gather


def _unfold_two(wi):
    m, d = wi.shape
    dw = d // 2
    w = lax.bitcast_convert_type(wi, jnp.uint32)
    lo = lax.bitcast_convert_type(w << 16, jnp.float32)
    hi = lax.bitcast_convert_type(w & jnp.uint32(0xFFFF0000), jnp.float32)
    even = jnp.concatenate([lo[:, :dw], hi[:, :dw]], axis=1)
    odd = jnp.concatenate([lo[:, dw:], hi[:, dw:]], axis=1)
    return even, odd


def _att_combine(center, aggc, aggd, aggg, ws):
    x = jnp.concatenate([center, aggc, aggd, aggg], axis=1)
    s = jnp.dot(x, ws, preferred_element_type=jnp.float32)
    s = jnp.where(s >= 0, s, 0.2 * s)
    m = jnp.max(s, axis=1, keepdims=True)
    e = jnp.exp(s - m)
    a = e / jnp.sum(e, axis=1, keepdims=True)
    return (a[:, 0:1] * center + a[:, 1:2] * aggc
            + a[:, 2:3] * aggd + a[:, 3:4] * aggg)


def _combine_body(s_per, d, center_ref, sc_ref, sd_ref, sg_ref, ws_ref, o_ref):
    inv = 1.0 / s_per
    ctr_e, ctr_o = _unfold_two(center_ref[...])
    c_e, c_o = _unfold_two(sc_ref[...])
    d_e, d_o = _unfold_two(sd_ref[...])
    g_e, g_o = _unfold_two(sg_ref[...])
    ws = ws_ref[...]
    out_e = _att_combine(ctr_e, c_e * inv, d_e * inv, g_e * inv, ws)
    out_o = _att_combine(ctr_o, c_o * inv, d_o * inv, g_o * inv, ws)
    o_ref[...] = jnp.concatenate([out_e, out_o], axis=1)


def _combine(center, sum_c, sum_d, sum_g, att, s_per):
    Bn, dw = center.shape
    d = 2 * dw
    blk = 1024
    assert Bn % (2 * blk) == 0
    a1 = att[:, :d].T
    ws = jnp.zeros((4 * d, 4), jnp.float32)
    for t in range(4):
        ws = ws.at[t * d:(t + 1) * d, t].set(att[t, d:])
    ws = ws.at[:d, :].add(a1)
    fold = lambda x: x.reshape(Bn // 2, d)
    body = functools.partial(_combine_body, s_per, d)
    out = pl.pallas_call(
        body,
        grid=(Bn // (2 * blk),),
        in_specs=[pl.BlockSpec((blk, d), lambda i: (i, 0))] * 4
        + [pl.BlockSpec((4 * d, 4), lambda i: (0, 0))],
        out_specs=pl.BlockSpec((blk, 2 * d), lambda i: (i, 0)),
        out_shape=jax.ShapeDtypeStruct((Bn // 2, 2 * d), jnp.float32),
    )(fold(center), fold(sum_c), fold(sum_d), fold(sum_g), ws)
    return out.reshape(Bn, d)


def kernel(id_batch, neigh_cell, neigh_drug, neigh_gene, drug_features,
           gene_features, cell_table, W_drug, b_drug, W_gene, b_gene, att):
    Bn, s_per = neigh_cell.shape
    d = W_gene.shape[1]
    sc_cell = _make_sc_gather(Bn, d, s_per, [{"reduce": True}])
    pcell = _pack_table(cell_table)
    ncell_idx = _remap_idx(neigh_cell.reshape(-1), cell_table.shape[0])
    (sum_c,) = sc_cell(pcell, ncell_idx)
    pcell_b, ncell_b, gene_features, drug_features = lax.optimization_barrier(
        (pcell, ncell_idx, gene_features, drug_features))
    del pcell_b, ncell_b
    pgene = _project(gene_features, W_gene, b_gene)
    pdrug = _project(drug_features, W_drug, b_drug)
    sum_c_b, pgene, pdrug = lax.optimization_barrier((sum_c, pgene, pdrug))
    del sum_c_b
    n_gene = gene_features.shape[0]
    n_drug = drug_features.shape[0]
    sc_main = _make_sc_gather(
        Bn, d, s_per,
        [{"reduce": False}, {"reduce": True}, {"reduce": True}])
    center, sum_d, sum_g = sc_main(
        pgene, pdrug, pgene,
        _remap_idx(id_batch, n_gene),
        _remap_idx(neigh_drug.reshape(-1), n_drug),
        _remap_idx(neigh_gene.reshape(-1), n_gene))
    return _combine(center, sum_c, sum_d, sum_g, att, s_per)


# speedup vs baseline: 1.3133x; 1.3133x over previous
import functools

import jax
import jax.numpy as jnp
from jax import lax
from jax.experimental import pallas as pl
from jax.experimental.pallas import tpu as pltpu
from jax.experimental.pallas import tpu_sc as plsc


def _pack_rows(y):
    w = lax.bitcast_convert_type(y, jnp.uint32)
    r = (w + 0x7FFF + ((w >> 16) & 1)) >> 16
    dw = y.shape[1] // 2
    packed = r[:, :dw] | (r[:, dw:] << 16)
    return lax.bitcast_convert_type(packed, jnp.int32)


def _unpack_rows(wi):
    w = lax.bitcast_convert_type(wi, jnp.uint32)
    lo = lax.bitcast_convert_type(w << 16, jnp.float32)
    hi = lax.bitcast_convert_type(w & jnp.uint32(0xFFFF0000), jnp.float32)
    return jnp.concatenate([lo, hi], axis=1)


def _proj_body(xa_ref, xb_ref, w_ref, b_ref, o_ref):
    w = w_ref[...]
    b = b_ref[...]
    ya = jnp.dot(xa_ref[...], w, preferred_element_type=jnp.float32) + b
    yb = jnp.dot(xb_ref[...], w, preferred_element_type=jnp.float32) + b
    o_ref[...] = jnp.concatenate([_pack_rows(ya), _pack_rows(yb)], axis=1)


def _project(x, w, b):
    n, fdim = x.shape
    d = w.shape[1]
    blk = 1000
    nb = n // 2 // blk
    assert (n // 2) % blk == 0
    folded = pl.pallas_call(
        _proj_body,
        grid=(nb,),
        in_specs=[
            pl.BlockSpec((blk, fdim), lambda i: (i, 0)),
            pl.BlockSpec((blk, fdim), lambda i, nb_=nb: (i + nb_, 0)),
            pl.BlockSpec((fdim, d), lambda i: (0, 0)),
            pl.BlockSpec((1, d), lambda i: (0, 0)),
        ],
        out_specs=pl.BlockSpec((blk, d), lambda i: (i, 0)),
        out_shape=jax.ShapeDtypeStruct((n // 2, d), jnp.int32),
    )(x, x, w, b.reshape(1, d))
    return folded.reshape(n, d // 2)


def _cast_body(x_ref, o_ref):
    x = x_ref[...]
    h = x.shape[0] // 2
    o_ref[...] = jnp.concatenate(
        [_pack_rows(x[:h]), _pack_rows(x[h:])], axis=1)


def _pack_table(x):
    n, d = x.shape
    folded = pl.pallas_call(
        _cast_body,
        out_shape=jax.ShapeDtypeStruct((n // 2, d), jnp.int32),
    )(x)
    return folded.reshape(n, d // 2)


def _remap_idx(idx, n):
    h = n // 2
    idx = idx.astype(jnp.int32)
    return jnp.where(idx < h, 2 * idx, 2 * idx - (n - 1))


def _make_sc_gather(Bn, d, s_per, streams):
    info = plsc.get_sparse_core_info()
    NW = info.num_cores * info.num_subcores
    CB = 16
    G = CB * s_per
    GH = G // 2
    rows_w = Bn // NW
    nchunks = rows_w // CB
    assert Bn % (NW * CB) == 0 and nchunks % 2 == 0
    mesh = plsc.VectorSubcoreMesh(core_axis_name="c", subcore_axis_name="s")
    ns = len(streams)

    dw = d // 2
    idx_scratch = [
        pltpu.VMEM((rows_w * (s_per if st["reduce"] else 1),), jnp.int32)
        for st in streams
    ]
    row_scratch = [
        pltpu.VMEM(((G if st["reduce"] else CB), dw), jnp.int32)
        for st in streams
    ] * 2
    acc_scratch = [pltpu.VMEM((CB, dw), jnp.int32)
                   for st in streams if st["reduce"]] * 2
    nred = sum(1 for st in streams if st["reduce"])

    @functools.partial(
        pl.kernel,
        mesh=mesh,
        out_type=[jax.ShapeDtypeStruct((Bn, dw), jnp.int32)] * ns,
        scratch_types=idx_scratch + row_scratch + acc_scratch
        + [pltpu.SemaphoreType.DMA] * 4,
        compiler_params=pltpu.CompilerParams(use_tc_tiling_on_sc=False,
                                             needs_layout_passes=False),
    )
    def sc_gather(*refs):
        tabs = refs[:ns]
        idxs = refs[ns:2 * ns]
        outs = refs[2 * ns:3 * ns]
        k = 3 * ns
        ix = refs[k:k + ns]
        rows = (refs[k + ns:k + 2 * ns], refs[k + 2 * ns:k + 3 * ns])
        k2 = k + 3 * ns
        accs = (refs[k2:k2 + nred], refs[k2 + nred:k2 + 2 * nred])
        sem_g = refs[k2 + 2 * nred:k2 + 2 * nred + 2]
        sem_o = refs[k2 + 2 * nred + 2:k2 + 2 * nred + 4]

        wid = lax.axis_index("s") * info.num_cores + lax.axis_index("c")
        base = wid * rows_w

        for t, st in enumerate(streams):
            rep = s_per if st["reduce"] else 1
            pltpu.sync_copy(idxs[t].at[pl.ds(base * rep, rows_w * rep)],
                            ix[t])

        def fire_gathers(c, s):
            for t, st in enumerate(streams):
                if st["reduce"]:
                    o = pl.multiple_of(c * G, 8)
                    pltpu.async_copy(
                        tabs[t].at[ix[t].at[pl.ds(o, GH)]],
                        rows[s][t].at[pl.ds(0, GH)], sem_g[s])
                    pltpu.async_copy(
                        tabs[t].at[ix[t].at[pl.ds(o + GH, GH)]],
                        rows[s][t].at[pl.ds(GH, GH)], sem_g[s])
                else:
                    o = pl.multiple_of(c * CB, CB)
                    pltpu.async_copy(tabs[t].at[ix[t].at[pl.ds(o, CB)]],
                                     rows[s][t], sem_g[s])

        def wait_gathers(s):
            for t, st in enumerate(streams):
                if st["reduce"]:
                    pltpu.make_async_copy(tabs[t].at[pl.ds(0, GH)],
                                          rows[s][t].at[pl.ds(0, GH)],
                                          sem_g[s]).wait()
                    pltpu.make_async_copy(tabs[t].at[pl.ds(0, GH)],
                                          rows[s][t].at[pl.ds(GH, GH)],
                                          sem_g[s]).wait()
                else:
                    pltpu.make_async_copy(tabs[t].at[pl.ds(0, CB)],
                                          rows[s][t], sem_g[s]).wait()

        def fire_out(c, s):
            off = pl.multiple_of(base + c * CB, CB)
            r = 0
            for t, st in enumerate(streams):
                src = accs[s][r] if st["reduce"] else rows[s][t]
                if st["reduce"]:
                    r += 1
                pltpu.async_copy(src, outs[t].at[pl.ds(off, CB)], sem_o[s])

        def drain_out(s):
            r = 0
            for t, st in enumerate(streams):
                dst = accs[s][r] if st["reduce"] else rows[s][t]
                if st["reduce"]:
                    r += 1
                pltpu.make_async_copy(outs[t].at[pl.ds(0, CB)], dst,
                                      sem_o[s]).wait()

        def reduce(rows_t, acc_t):
            bf = jnp.bfloat16

            def per_center(i, carry):
                rbase = i * s_per
                for g in range(dw // 16):
                    sl = pl.ds(g * 16, 16)
                    vals = [plsc.bitcast(rows_t[rbase + j, sl], bf)
                            for j in range(s_per)]
                    while len(vals) > 1:
                        nxt = [vals[k] + vals[k + 1]
                               for k in range(0, len(vals) - 1, 2)]
                        if len(vals) % 2:
                            nxt.append(vals[-1])
                        vals = nxt
                    acc_t[i, sl] = plsc.bitcast(vals[0], jnp.int32)
                return carry
            lax.fori_loop(0, CB, per_center, 0)

        def reduce_all(s):
            r = 0
            for t, st in enumerate(streams):
                if st["reduce"]:
                    reduce(rows[s][t], accs[s][r])
                    r += 1

        fire_gathers(0, 0)

        def step(cc, carry):
            for s in (0, 1):
                c = cc * 2 + s
                if s == 0:
                    @pl.when(cc > 0)
                    def _():
                        drain_out(1)
                    fire_gathers(c + 1, 1)
                else:
                    drain_out(0)

                    @pl.when(cc < nchunks // 2 - 1)
                    def _():
                        fire_gathers(c + 1, 0)
                wait_gathers(s)
                reduce_all(s)
                fire_out(c, s)
            return carry

        lax.fori_loop(0, nchunks // 2, step, 0)
        drain_out(1)

    return sc_gather


def _unfold_two(wi):
    m, d = wi.shape
    dw = d // 2
    w = lax.bitcast_convert_type(wi, jnp.uint32)
    lo = lax.bitcast_convert_type(w << 16, jnp.float32)
    hi = lax.bitcast_convert_type(w & jnp.uint32(0xFFFF0000), jnp.float32)
    even = jnp.concatenate([lo[:, :dw], hi[:, :dw]], axis=1)
    odd = jnp.concatenate([lo[:, dw:], hi[:, dw:]], axis=1)
    return even, odd


def _att_combine(center, aggc, aggd, aggg, ws):
    x = jnp.concatenate([center, aggc, aggd, aggg], axis=1)
    s = jnp.dot(x, ws, preferred_element_type=jnp.float32)
    s = jnp.where(s >= 0, s, 0.2 * s)
    m = jnp.max(s, axis=1, keepdims=True)
    e = jnp.exp(s - m)
    a = e / jnp.sum(e, axis=1, keepdims=True)
    return (a[:, 0:1] * center + a[:, 1:2] * aggc
            + a[:, 2:3] * aggd + a[:, 3:4] * aggg)


def _combine_body(s_per, d, center_ref, sc_ref, sd_ref, sg_ref, ws_ref, o_ref):
    inv = 1.0 / s_per
    ctr_e, ctr_o = _unfold_two(center_ref[...])
    c_e, c_o = _unfold_two(sc_ref[...])
    d_e, d_o = _unfold_two(sd_ref[...])
    g_e, g_o = _unfold_two(sg_ref[...])
    ws = ws_ref[...]
    out_e = _att_combine(ctr_e, c_e * inv, d_e * inv, g_e * inv, ws)
    out_o = _att_combine(ctr_o, c_o * inv, d_o * inv, g_o * inv, ws)
    o_ref[...] = jnp.concatenate([out_e, out_o], axis=1)


def _combine(center, sum_c, sum_d, sum_g, att, s_per):
    Bn, dw = center.shape
    d = 2 * dw
    blk = 1024
    assert Bn % (2 * blk) == 0
    a1 = att[:, :d].T
    ws = jnp.zeros((4 * d, 4), jnp.float32)
    for t in range(4):
        ws = ws.at[t * d:(t + 1) * d, t].set(att[t, d:])
    ws = ws.at[:d, :].add(a1)
    fold = lambda x: x.reshape(Bn // 2, d)
    body = functools.partial(_combine_body, s_per, d)
    out = pl.pallas_call(
        body,
        grid=(Bn // (2 * blk),),
        in_specs=[pl.BlockSpec((blk, d), lambda i: (i, 0))] * 4
        + [pl.BlockSpec((4 * d, 4), lambda i: (0, 0))],
        out_specs=pl.BlockSpec((blk, 2 * d), lambda i: (i, 0)),
        out_shape=jax.ShapeDtypeStruct((Bn // 2, 2 * d), jnp.float32),
    )(fold(center), fold(sum_c), fold(sum_d), fold(sum_g), ws)
    return out.reshape(Bn, d)


def kernel(id_batch, neigh_cell, neigh_drug, neigh_gene, drug_features,
           gene_features, cell_table, W_drug, b_drug, W_gene, b_gene, att):
    Bn, s_per = neigh_cell.shape
    d = W_gene.shape[1]
    pcell = _pack_table(cell_table)
    pgene = _project(gene_features, W_gene, b_gene)
    pdrug = _project(drug_features, W_drug, b_drug)
    n_gene = gene_features.shape[0]
    n_drug = drug_features.shape[0]
    sc_all = _make_sc_gather(
        Bn, d, s_per,
        [{"reduce": False}, {"reduce": True}, {"reduce": True},
         {"reduce": True}])
    center, sum_c, sum_d, sum_g = sc_all(
        pgene, pcell, pdrug, pgene,
        _remap_idx(id_batch, n_gene),
        _remap_idx(neigh_cell.reshape(-1), cell_table.shape[0]),
        _remap_idx(neigh_drug.reshape(-1), n_drug),
        _remap_idx(neigh_gene.reshape(-1), n_gene))
    return _combine(center, sum_c, sum_d, sum_g, att, s_per)


# speedup vs baseline: 1.5343x; 1.1683x over previous
import functools

import jax
import jax.numpy as jnp
from jax import lax
from jax.experimental import pallas as pl
from jax.experimental.pallas import tpu as pltpu
from jax.experimental.pallas import tpu_sc as plsc


def _pack_rows(y):
    w = lax.bitcast_convert_type(y, jnp.uint32)
    r = (w + 0x7FFF + ((w >> 16) & 1)) >> 16
    dw = y.shape[1] // 2
    packed = r[:, :dw] | (r[:, dw:] << 16)
    return lax.bitcast_convert_type(packed, jnp.int32)


def _unpack_rows(wi):
    w = lax.bitcast_convert_type(wi, jnp.uint32)
    lo = lax.bitcast_convert_type(w << 16, jnp.float32)
    hi = lax.bitcast_convert_type(w & jnp.uint32(0xFFFF0000), jnp.float32)
    return jnp.concatenate([lo, hi], axis=1)


def _proj_body(xa_ref, xb_ref, w_ref, b_ref, o_ref):
    w = w_ref[...]
    b = b_ref[...]
    ya = jnp.dot(xa_ref[...], w, preferred_element_type=jnp.float32) + b
    yb = jnp.dot(xb_ref[...], w, preferred_element_type=jnp.float32) + b
    o_ref[...] = jnp.concatenate([_pack_rows(ya), _pack_rows(yb)], axis=1)


def _project(x, w, b):
    n, fdim = x.shape
    d = w.shape[1]
    blk = 1000
    nb = n // 2 // blk
    assert (n // 2) % blk == 0
    folded = pl.pallas_call(
        _proj_body,
        grid=(nb,),
        in_specs=[
            pl.BlockSpec((blk, fdim), lambda i: (i, 0)),
            pl.BlockSpec((blk, fdim), lambda i, nb_=nb: (i + nb_, 0)),
            pl.BlockSpec((fdim, d), lambda i: (0, 0)),
            pl.BlockSpec((1, d), lambda i: (0, 0)),
        ],
        out_specs=pl.BlockSpec((blk, d), lambda i: (i, 0)),
        out_shape=jax.ShapeDtypeStruct((n // 2, d), jnp.int32),
    )(x, x, w, b.reshape(1, d))
    return folded.reshape(n, d // 2)


def _cast_body(x_ref, o_ref):
    x = x_ref[...]
    h = x.shape[0] // 2
    o_ref[...] = jnp.concatenate(
        [_pack_rows(x[:h]), _pack_rows(x[h:])], axis=1)


def _pack_table(x):
    n, d = x.shape
    folded = pl.pallas_call(
        _cast_body,
        out_shape=jax.ShapeDtypeStruct((n // 2, d), jnp.int32),
    )(x)
    return folded.reshape(n, d // 2)


def _remap_idx(idx, n):
    h = n // 2
    idx = idx.astype(jnp.int32)
    return jnp.where(idx < h, 2 * idx, 2 * idx - (n - 1))


def _make_sc_gather(Bn, d, s_per, streams):
    info = plsc.get_sparse_core_info()
    NW = info.num_cores * info.num_subcores
    CB = 16
    G = CB * s_per
    GH = G // 2
    rows_w = Bn // NW
    nchunks = rows_w // CB
    assert Bn % (NW * CB) == 0 and nchunks % 2 == 0
    mesh = plsc.VectorSubcoreMesh(core_axis_name="c", subcore_axis_name="s")
    ns = len(streams)

    dw = d // 2
    idx_scratch = [
        pltpu.VMEM((rows_w * (s_per if st["reduce"] else 1),), jnp.int32)
        for st in streams
    ]
    row_scratch = [
        pltpu.VMEM(((G if st["reduce"] else CB), dw), jnp.int32)
        for st in streams
    ] * 2
    acc_scratch = [pltpu.VMEM((CB, dw), jnp.int32)
                   for st in streams if st["reduce"]] * 2
    nred = sum(1 for st in streams if st["reduce"])

    @functools.partial(
        pl.kernel,
        mesh=mesh,
        out_type=[jax.ShapeDtypeStruct((Bn, dw), jnp.int32)] * ns,
        scratch_types=idx_scratch + row_scratch + acc_scratch
        + [pltpu.SemaphoreType.DMA] * 4,
        compiler_params=pltpu.CompilerParams(use_tc_tiling_on_sc=False,
                                             needs_layout_passes=False),
    )
    def sc_gather(*refs):
        tabs = refs[:ns]
        idxs = refs[ns:2 * ns]
        outs = refs[2 * ns:3 * ns]
        k = 3 * ns
        ix = refs[k:k + ns]
        rows = (refs[k + ns:k + 2 * ns], refs[k + 2 * ns:k + 3 * ns])
        k2 = k + 3 * ns
        accs = (refs[k2:k2 + nred], refs[k2 + nred:k2 + 2 * nred])
        sem_g = refs[k2 + 2 * nred:k2 + 2 * nred + 2]
        sem_o = refs[k2 + 2 * nred + 2:k2 + 2 * nred + 4]

        wid = lax.axis_index("s") * info.num_cores + lax.axis_index("c")
        base = wid * rows_w

        for t, st in enumerate(streams):
            rep = s_per if st["reduce"] else 1
            pltpu.sync_copy(idxs[t].at[pl.ds(base * rep, rows_w * rep)],
                            ix[t])
            half = st["half"]

            def remap_grp(g, carry, t=t, half=half, rep=rep):
                sl = pl.ds(g * 16, 16)
                v = ix[t][sl]
                ix[t][sl] = jnp.where(v < half, 2 * v,
                                      2 * v - (2 * half - 1))
                return carry
            lax.fori_loop(0, rows_w * rep // 16, remap_grp, 0)

        def fire_gathers(c, s):
            for t, st in enumerate(streams):
                if st["reduce"]:
                    o = pl.multiple_of(c * G, 8)
                    pltpu.async_copy(
                        tabs[t].at[ix[t].at[pl.ds(o, GH)]],
                        rows[s][t].at[pl.ds(0, GH)], sem_g[s])
                    pltpu.async_copy(
                        tabs[t].at[ix[t].at[pl.ds(o + GH, GH)]],
                        rows[s][t].at[pl.ds(GH, GH)], sem_g[s])
                else:
                    o = pl.multiple_of(c * CB, CB)
                    pltpu.async_copy(tabs[t].at[ix[t].at[pl.ds(o, CB)]],
                                     rows[s][t], sem_g[s])

        def wait_gathers(s):
            for t, st in enumerate(streams):
                if st["reduce"]:
                    pltpu.make_async_copy(tabs[t].at[pl.ds(0, GH)],
                                          rows[s][t].at[pl.ds(0, GH)],
                                          sem_g[s]).wait()
                    pltpu.make_async_copy(tabs[t].at[pl.ds(0, GH)],
                                          rows[s][t].at[pl.ds(GH, GH)],
                                          sem_g[s]).wait()
                else:
                    pltpu.make_async_copy(tabs[t].at[pl.ds(0, CB)],
                                          rows[s][t], sem_g[s]).wait()

        def fire_out(c, s):
            off = pl.multiple_of(base + c * CB, CB)
            r = 0
            for t, st in enumerate(streams):
                src = accs[s][r] if st["reduce"] else rows[s][t]
                if st["reduce"]:
                    r += 1
                pltpu.async_copy(src, outs[t].at[pl.ds(off, CB)], sem_o[s])

        def drain_out(s):
            r = 0
            for t, st in enumerate(streams):
                dst = accs[s][r] if st["reduce"] else rows[s][t]
                if st["reduce"]:
                    r += 1
                pltpu.make_async_copy(outs[t].at[pl.ds(0, CB)], dst,
                                      sem_o[s]).wait()

        def reduce(rows_t, acc_t):
            bf = jnp.bfloat16

            def per_center(i, carry):
                rbase = i * s_per
                for g in range(dw // 16):
                    sl = pl.ds(g * 16, 16)
                    vals = [plsc.bitcast(rows_t[rbase + j, sl], bf)
                            for j in range(s_per)]
                    while len(vals) > 1:
                        nxt = [vals[k] + vals[k + 1]
                               for k in range(0, len(vals) - 1, 2)]
                        if len(vals) % 2:
                            nxt.append(vals[-1])
                        vals = nxt
                    acc_t[i, sl] = plsc.bitcast(vals[0], jnp.int32)
                return carry
            lax.fori_loop(0, CB, per_center, 0)

        def reduce_all(s):
            r = 0
            for t, st in enumerate(streams):
                if st["reduce"]:
                    reduce(rows[s][t], accs[s][r])
                    r += 1

        fire_gathers(0, 0)

        def step(cc, carry):
            for s in (0, 1):
                c = cc * 2 + s
                if s == 0:
                    @pl.when(cc > 0)
                    def _():
                        drain_out(1)
                    fire_gathers(c + 1, 1)
                else:
                    drain_out(0)

                    @pl.when(cc < nchunks // 2 - 1)
                    def _():
                        fire_gathers(c + 1, 0)
                wait_gathers(s)
                reduce_all(s)
                fire_out(c, s)
            return carry

        lax.fori_loop(0, nchunks // 2, step, 0)
        drain_out(1)

    return sc_gather


def _unfold_two(wi):
    m, d = wi.shape
    dw = d // 2
    w = lax.bitcast_convert_type(wi, jnp.uint32)
    lo = lax.bitcast_convert_type(w << 16, jnp.float32)
    hi = lax.bitcast_convert_type(w & jnp.uint32(0xFFFF0000), jnp.float32)
    even = jnp.concatenate([lo[:, :dw], hi[:, :dw]], axis=1)
    odd = jnp.concatenate([lo[:, dw:], hi[:, dw:]], axis=1)
    return even, odd


def _att_combine(center, aggc, aggd, aggg, ws):
    x = jnp.concatenate([center, aggc, aggd, aggg], axis=1)
    s = jnp.dot(x, ws, preferred_element_type=jnp.float32)
    s = jnp.where(s >= 0, s, 0.2 * s)
    m = jnp.max(s, axis=1, keepdims=True)
    e = jnp.exp(s - m)
    a = e / jnp.sum(e, axis=1, keepdims=True)
    return (a[:, 0:1] * center + a[:, 1:2] * aggc
            + a[:, 2:3] * aggd + a[:, 3:4] * aggg)


def _combine_body(s_per, d, center_ref, sc_ref, sd_ref, sg_ref, ws_ref, o_ref):
    inv = 1.0 / s_per
    ctr_e, ctr_o = _unfold_two(center_ref[...])
    c_e, c_o = _unfold_two(sc_ref[...])
    d_e, d_o = _unfold_two(sd_ref[...])
    g_e, g_o = _unfold_two(sg_ref[...])
    ws = ws_ref[...]
    out_e = _att_combine(ctr_e, c_e * inv, d_e * inv, g_e * inv, ws)
    out_o = _att_combine(ctr_o, c_o * inv, d_o * inv, g_o * inv, ws)
    o_ref[...] = jnp.concatenate([out_e, out_o], axis=1)


def _combine(center, sum_c, sum_d, sum_g, att, s_per):
    Bn, dw = center.shape
    d = 2 * dw
    blk = 1024
    assert Bn % (2 * blk) == 0
    a1 = att[:, :d].T
    ws = jnp.zeros((4 * d, 4), jnp.float32)
    for t in range(4):
        ws = ws.at[t * d:(t + 1) * d, t].set(att[t, d:])
    ws = ws.at[:d, :].add(a1)
    fold = lambda x: x.reshape(Bn // 2, d)
    body = functools.partial(_combine_body, s_per, d)
    out = pl.pallas_call(
        body,
        grid=(Bn // (2 * blk),),
        in_specs=[pl.BlockSpec((blk, d), lambda i: (i, 0))] * 4
        + [pl.BlockSpec((4 * d, 4), lambda i: (0, 0))],
        out_specs=pl.BlockSpec((blk, 2 * d), lambda i: (i, 0)),
        out_shape=jax.ShapeDtypeStruct((Bn // 2, 2 * d), jnp.float32),
    )(fold(center), fold(sum_c), fold(sum_d), fold(sum_g), ws)
    return out.reshape(Bn, d)


def kernel(id_batch, neigh_cell, neigh_drug, neigh_gene, drug_features,
           gene_features, cell_table, W_drug, b_drug, W_gene, b_gene, att):
    Bn, s_per = neigh_cell.shape
    d = W_gene.shape[1]
    pcell = _pack_table(cell_table)
    pgene = _project(gene_features, W_gene, b_gene)
    pdrug = _project(drug_features, W_drug, b_drug)
    n_gene = gene_features.shape[0]
    n_drug = drug_features.shape[0]
    sc_all = _make_sc_gather(
        Bn, d, s_per,
        [{"reduce": False, "half": n_gene // 2},
         {"reduce": True, "half": cell_table.shape[0] // 2},
         {"reduce": True, "half": n_drug // 2},
         {"reduce": True, "half": n_gene // 2}])
    center, sum_c, sum_d, sum_g = sc_all(
        pgene, pcell, pdrug, pgene,
        id_batch.astype(jnp.int32),
        neigh_cell.reshape(-1).astype(jnp.int32),
        neigh_drug.reshape(-1).astype(jnp.int32),
        neigh_gene.reshape(-1).astype(jnp.int32))
    return _combine(center, sum_c, sum_d, sum_g, att, s_per)


# speedup vs baseline: 1.6649x; 1.0851x over previous
import functools

import jax
import jax.numpy as jnp
from jax import lax
from jax.experimental import pallas as pl
from jax.experimental.pallas import tpu as pltpu
from jax.experimental.pallas import tpu_sc as plsc


def _pack_rows(y):
    w = lax.bitcast_convert_type(y, jnp.uint32)
    r = (w + 0x7FFF + ((w >> 16) & 1)) >> 16
    dw = y.shape[1] // 2
    packed = r[:, :dw] | (r[:, dw:] << 16)
    return lax.bitcast_convert_type(packed, jnp.int32)


def _unpack_rows(wi):
    w = lax.bitcast_convert_type(wi, jnp.uint32)
    lo = lax.bitcast_convert_type(w << 16, jnp.float32)
    hi = lax.bitcast_convert_type(w & jnp.uint32(0xFFFF0000), jnp.float32)
    return jnp.concatenate([lo, hi], axis=1)


def _proj_body(xa_ref, xb_ref, w_ref, b_ref, o_ref):
    w = w_ref[...]
    b = b_ref[...]
    ya = jnp.dot(xa_ref[...], w, preferred_element_type=jnp.float32) + b
    yb = jnp.dot(xb_ref[...], w, preferred_element_type=jnp.float32) + b
    o_ref[...] = jnp.concatenate([_pack_rows(ya), _pack_rows(yb)], axis=1)


def _project(x, w, b):
    n, fdim = x.shape
    d = w.shape[1]
    blk = 1000
    nb = n // 2 // blk
    assert (n // 2) % blk == 0
    folded = pl.pallas_call(
        _proj_body,
        grid=(nb,),
        in_specs=[
            pl.BlockSpec((blk, fdim), lambda i: (i, 0)),
            pl.BlockSpec((blk, fdim), lambda i, nb_=nb: (i + nb_, 0)),
            pl.BlockSpec((fdim, d), lambda i: (0, 0)),
            pl.BlockSpec((1, d), lambda i: (0, 0)),
        ],
        out_specs=pl.BlockSpec((blk, d), lambda i: (i, 0)),
        out_shape=jax.ShapeDtypeStruct((n // 2, d), jnp.int32),
    )(x, x, w, b.reshape(1, d))
    return folded.reshape(n, d // 2)


def _cast_body(x_ref, o_ref):
    x = x_ref[...]
    h = x.shape[0] // 2
    o_ref[...] = jnp.concatenate(
        [_pack_rows(x[:h]), _pack_rows(x[h:])], axis=1)


def _pack_table(x):
    n, d = x.shape
    folded = pl.pallas_call(
        _cast_body,
        out_shape=jax.ShapeDtypeStruct((n // 2, d), jnp.int32),
    )(x)
    return folded.reshape(n, d // 2)


def _remap_idx(idx, n):
    h = n // 2
    idx = idx.astype(jnp.int32)
    return jnp.where(idx < h, 2 * idx, 2 * idx - (n - 1))


def _make_sc_gather(Bn, d, s_per, streams):
    info = plsc.get_sparse_core_info()
    NW = info.num_cores * info.num_subcores
    CB = 16
    G = CB * s_per
    GH = G // 2
    rows_w = Bn // NW
    nchunks = rows_w // CB
    assert Bn % (NW * CB) == 0 and nchunks % 2 == 0
    mesh = plsc.VectorSubcoreMesh(core_axis_name="c", subcore_axis_name="s")
    ns = len(streams)

    dw = d // 2
    idx_scratch = [
        pltpu.VMEM((rows_w * (s_per if st["reduce"] else 1),), jnp.int32)
        for st in streams
    ]
    row_scratch = [
        pltpu.VMEM(((G if st["reduce"] else CB), dw), jnp.int32)
        for st in streams
    ] * 2
    acc_scratch = [pltpu.VMEM((CB, dw), jnp.int32)
                   for st in streams if st["reduce"]] * 2
    nred = sum(1 for st in streams if st["reduce"])

    @functools.partial(
        pl.kernel,
        mesh=mesh,
        out_type=[jax.ShapeDtypeStruct((Bn, dw), jnp.int32)] * ns,
        scratch_types=idx_scratch + row_scratch + acc_scratch
        + [pltpu.SemaphoreType.DMA] * 4,
        compiler_params=pltpu.CompilerParams(use_tc_tiling_on_sc=False,
                                             needs_layout_passes=False),
    )
    def sc_gather(*refs):
        tabs = refs[:ns]
        idxs = refs[ns:2 * ns]
        outs = refs[2 * ns:3 * ns]
        k = 3 * ns
        ix = refs[k:k + ns]
        rows = (refs[k + ns:k + 2 * ns], refs[k + 2 * ns:k + 3 * ns])
        k2 = k + 3 * ns
        accs = (refs[k2:k2 + nred], refs[k2 + nred:k2 + 2 * nred])
        sem_g = refs[k2 + 2 * nred:k2 + 2 * nred + 2]
        sem_o = refs[k2 + 2 * nred + 2:k2 + 2 * nred + 4]

        wid = lax.axis_index("s") * info.num_cores + lax.axis_index("c")
        base = wid * rows_w

        for t, st in enumerate(streams):
            rep = s_per if st["reduce"] else 1
            pltpu.sync_copy(idxs[t].at[pl.ds(base * rep, rows_w * rep)],
                            ix[t])
            half = st["half"]

            def remap_grp(g, carry, t=t, half=half, rep=rep):
                sl = pl.ds(g * 16, 16)
                v = ix[t][sl]
                ix[t][sl] = jnp.where(v < half, 2 * v,
                                      2 * v - (2 * half - 1))
                return carry
            lax.fori_loop(0, rows_w * rep // 16, remap_grp, 0)

        def fire_gathers(c, s):
            for t, st in enumerate(streams):
                if st["reduce"]:
                    o = pl.multiple_of(c * G, 8)
                    pltpu.async_copy(
                        tabs[t].at[ix[t].at[pl.ds(o, GH)]],
                        rows[s][t].at[pl.ds(0, GH)], sem_g[s])
                    pltpu.async_copy(
                        tabs[t].at[ix[t].at[pl.ds(o + GH, GH)]],
                        rows[s][t].at[pl.ds(GH, GH)], sem_g[s])
                else:
                    o = pl.multiple_of(c * CB, CB)
                    pltpu.async_copy(tabs[t].at[ix[t].at[pl.ds(o, CB)]],
                                     rows[s][t], sem_g[s])

        def wait_gathers(s):
            for t, st in enumerate(streams):
                if st["reduce"]:
                    pltpu.make_async_copy(tabs[t].at[pl.ds(0, GH)],
                                          rows[s][t].at[pl.ds(0, GH)],
                                          sem_g[s]).wait()
                    pltpu.make_async_copy(tabs[t].at[pl.ds(0, GH)],
                                          rows[s][t].at[pl.ds(GH, GH)],
                                          sem_g[s]).wait()
                else:
                    pltpu.make_async_copy(tabs[t].at[pl.ds(0, CB)],
                                          rows[s][t], sem_g[s]).wait()

        def fire_out(c, s):
            off = pl.multiple_of(base + c * CB, CB)
            r = 0
            for t, st in enumerate(streams):
                src = accs[s][r] if st["reduce"] else rows[s][t]
                if st["reduce"]:
                    r += 1
                pltpu.async_copy(src, outs[t].at[pl.ds(off, CB)], sem_o[s])

        def drain_out(s):
            r = 0
            for t, st in enumerate(streams):
                dst = accs[s][r] if st["reduce"] else rows[s][t]
                if st["reduce"]:
                    r += 1
                pltpu.make_async_copy(outs[t].at[pl.ds(0, CB)], dst,
                                      sem_o[s]).wait()

        def reduce(rows_t, acc_t):
            bf = jnp.bfloat16

            def per_center(i, carry):
                rbase = i * s_per
                for g in range(dw // 16):
                    sl = pl.ds(g * 16, 16)
                    vals = [plsc.bitcast(rows_t[rbase + j, sl], bf)
                            for j in range(s_per)]
                    while len(vals) > 1:
                        nxt = [vals[k] + vals[k + 1]
                               for k in range(0, len(vals) - 1, 2)]
                        if len(vals) % 2:
                            nxt.append(vals[-1])
                        vals = nxt
                    acc_t[i, sl] = plsc.bitcast(vals[0], jnp.int32)
                return carry
            lax.fori_loop(0, CB, per_center, 0)

        def reduce_all(s):
            r = 0
            for t, st in enumerate(streams):
                if st["reduce"]:
                    reduce(rows[s][t], accs[s][r])
                    r += 1

        fire_gathers(0, 0)

        def step(cc, carry):
            for s in (0, 1):
                c = cc * 2 + s
                if s == 0:
                    @pl.when(cc > 0)
                    def _():
                        drain_out(1)
                    fire_gathers(c + 1, 1)
                else:
                    drain_out(0)

                    @pl.when(cc < nchunks // 2 - 1)
                    def _():
                        fire_gathers(c + 1, 0)
                wait_gathers(s)
                reduce_all(s)
                fire_out(c, s)
            return carry

        lax.fori_loop(0, nchunks // 2, step, 0)
        drain_out(1)

    return sc_gather


def _unfold_two(wi):
    m, d = wi.shape
    dw = d // 2
    w = lax.bitcast_convert_type(wi, jnp.uint32)
    lo = lax.bitcast_convert_type(w << 16, jnp.float32)
    hi = lax.bitcast_convert_type(w & jnp.uint32(0xFFFF0000), jnp.float32)
    even = jnp.concatenate([lo[:, :dw], hi[:, :dw]], axis=1)
    odd = jnp.concatenate([lo[:, dw:], hi[:, dw:]], axis=1)
    return even, odd


def _att_combine(center, aggc, aggd, aggg, ws):
    x = jnp.concatenate([center, aggc, aggd, aggg], axis=1)
    s = jnp.dot(x, ws, preferred_element_type=jnp.float32)
    s = jnp.where(s >= 0, s, 0.2 * s)
    m = jnp.max(s, axis=1, keepdims=True)
    e = jnp.exp(s - m)
    a = e / jnp.sum(e, axis=1, keepdims=True)
    return (a[:, 0:1] * center + a[:, 1:2] * aggc
            + a[:, 2:3] * aggd + a[:, 3:4] * aggg)


def _combine_body(s_per, d, center_ref, sc_ref, sd_ref, sg_ref, ws_ref, o_ref):
    inv = 1.0 / s_per
    ctr_e, ctr_o = _unfold_two(center_ref[...])
    c_e, c_o = _unfold_two(sc_ref[...])
    d_e, d_o = _unfold_two(sd_ref[...])
    g_e, g_o = _unfold_two(sg_ref[...])
    ws = ws_ref[...]
    out_e = _att_combine(ctr_e, c_e * inv, d_e * inv, g_e * inv, ws)
    out_o = _att_combine(ctr_o, c_o * inv, d_o * inv, g_o * inv, ws)
    o_ref[...] = jnp.concatenate([out_e, out_o], axis=1)


def _combine(center, sum_c, sum_d, sum_g, att, s_per):
    Bn, dw = center.shape
    d = 2 * dw
    blk = 1024
    assert Bn % (2 * blk) == 0
    a1 = att[:, :d].T
    ws = jnp.zeros((4 * d, 4), jnp.float32)
    for t in range(4):
        ws = ws.at[t * d:(t + 1) * d, t].set(att[t, d:])
    ws = ws.at[:d, :].add(a1)
    fold = lambda x: x.reshape(Bn // 2, d)
    body = functools.partial(_combine_body, s_per, d)
    out = pl.pallas_call(
        body,
        grid=(Bn // (2 * blk),),
        in_specs=[pl.BlockSpec((blk, d), lambda i: (i, 0))] * 4
        + [pl.BlockSpec((4 * d, 4), lambda i: (0, 0))],
        out_specs=pl.BlockSpec((blk, 2 * d), lambda i: (i, 0)),
        out_shape=jax.ShapeDtypeStruct((Bn // 2, 2 * d), jnp.float32),
    )(fold(center), fold(sum_c), fold(sum_d), fold(sum_g), ws)
    return out.reshape(Bn, d)


def kernel(id_batch, neigh_cell, neigh_drug, neigh_gene, drug_features,
           gene_features, cell_table, W_drug, b_drug, W_gene, b_gene, att):
    Bn, s_per = neigh_cell.shape
    d = W_gene.shape[1]
    pcell = _pack_table(cell_table)
    pgene = _project(gene_features, W_gene, b_gene)
    n_gene = gene_features.shape[0]
    n_drug = drug_features.shape[0]
    sc_a = _make_sc_gather(
        Bn, d, s_per,
        [{"reduce": False, "half": n_gene // 2},
         {"reduce": True, "half": cell_table.shape[0] // 2},
         {"reduce": True, "half": n_gene // 2}])
    center, sum_c, sum_g = sc_a(
        pgene, pcell, pgene,
        id_batch.astype(jnp.int32),
        neigh_cell.reshape(-1).astype(jnp.int32),
        neigh_gene.reshape(-1).astype(jnp.int32))
    pdrug = _project(drug_features, W_drug, b_drug)
    sc_b = _make_sc_gather(Bn, d, s_per,
                           [{"reduce": True, "half": n_drug // 2}])
    (sum_d,) = sc_b(pdrug, neigh_drug.reshape(-1).astype(jnp.int32))
    return _combine(center, sum_c, sum_d, sum_g, att, s_per)
